# baseline (speedup 1.0000x reference)
import functools

import jax
import jax.numpy as jnp
from jax import lax
from jax.experimental import pallas as pl
from jax.experimental.pallas import tpu as pltpu
from jax.experimental.pallas import tpu_sc as plsc

N = 10000
C = 256
CH = 128
NC = 2
NS = 16
EB = 64
ACC = 10112
BM = 2000
NBUF = 5
IBUF = 7
GLA = 2
ILA = 4
RC = 80


def _mm_body(x_ref, d_ref, w_ref, o_ref):
    xs = x_ref[...] * d_ref[...]
    o_ref[...] = lax.dot_general(
        xs, w_ref[...], (((1,), (0,)), ((), ())),
        preferred_element_type=jnp.float32,
        precision=lax.Precision.DEFAULT)


def _matmul_scaled(x, dis, W):
    return pl.pallas_call(
        _mm_body,
        grid=(N // BM, NC),
        in_specs=[
            pl.BlockSpec((BM, C), lambda i, c: (i, 0)),
            pl.BlockSpec((BM, 1), lambda i, c: (i, 0)),
            pl.BlockSpec((C, CH), lambda i, c: (0, c)),
        ],
        out_specs=pl.BlockSpec((BM, CH), lambda i, c: (c * (N // BM) + i, 0)),
        out_shape=jax.ShapeDtypeStruct((NC * N, CH), jnp.float32),
    )(x, dis.reshape(N, 1), W)


def _make_sc_spmm(e_pad):
    per_tile = e_pad // NS
    nb = per_tile // EB
    mesh = plsc.VectorSubcoreMesh(core_axis_name="c", subcore_axis_name="s")

    @functools.partial(
        pl.kernel,
        out_type=jax.ShapeDtypeStruct((N, C), jnp.float32),
        mesh=mesh,
        scratch_types=[
            pltpu.VMEM((IBUF, EB), jnp.int32),
            pltpu.VMEM((IBUF, EB), jnp.int32),
            pltpu.VMEM((NBUF * EB, CH), jnp.float32),
            pltpu.VMEM((RC,), jnp.float32),
            pltpu.VMEM_SHARED((ACC, CH), jnp.float32),
            pltpu.SemaphoreType.DMA,
            pltpu.SemaphoreType.DMA,
            pltpu.SemaphoreType.DMA,
        ],
    )
    def sc_spmm(y_hbm, src_hbm, dst_hbm, dis_hbm, out_hbm,
                src_v, dst_v, rows_v, dis_c, acc_sh, sem_g, sem_s, sem_i):
        cid = lax.axis_index("c")
        sid = lax.axis_index("s")

        ebase = sid * per_tile
        soff = cid * e_pad
        idx, gd, sd = {}, {}, {}

        def start_idx(b):
            o = ebase + b * EB
            idx[b] = (
                pltpu.async_copy(src_hbm.at[pl.ds(soff + o, EB)],
                                 src_v.at[b % IBUF], sem_i),
                pltpu.async_copy(dst_hbm.at[pl.ds(o, EB)],
                                 dst_v.at[b % IBUF], sem_i))

        def start_gather(b):
            gd[b] = pltpu.async_copy(
                y_hbm.at[src_v.at[b % IBUF]],
                rows_v.at[pl.ds((b % NBUF) * EB, EB)], sem_g)

        def start_scatter(b):
            sd[b] = pltpu.async_copy(
                rows_v.at[pl.ds((b % NBUF) * EB, EB)],
                acc_sh.at[dst_v.at[b % IBUF]], sem_s, add=True)

        ZR = 128
        def zrow(r, carry):
            for k in range(CH // 16):
                rows_v[r, pl.ds(k * 16, 16)] = jnp.zeros((16,), jnp.float32)
            return carry
        lax.fori_loop(0, ZR, zrow, 0)
        arows = ACC // NS
        for j in range(arows // ZR):
            pltpu.sync_copy(rows_v.at[pl.ds(0, ZR)],
                            acc_sh.at[pl.ds(sid * arows + j * ZR, ZR)])
        rem = arows - (arows // ZR) * ZR
        pltpu.sync_copy(rows_v.at[pl.ds(0, rem)],
                        acc_sh.at[pl.ds(sid * arows + (arows // ZR) * ZR, rem)])

        for b in range(min(ILA, nb)):
            start_idx(b)
        for b in range(min(GLA, nb)):
            idx[b][0].wait()
            idx[b][1].wait()
            start_gather(b)
        plsc.subcore_barrier()

        for b in range(nb):
            if b >= 3:
                sd[b - 3].wait()
            if b + GLA < nb:
                idx[b + GLA][0].wait()
                idx[b + GLA][1].wait()
                start_gather(b + GLA)
            if b + ILA < nb:
                start_idx(b + ILA)
            gd[b].wait()
            start_scatter(b)
        for b in range(max(0, nb - 3), nb):
            sd[b].wait()

        plsc.subcore_barrier()

        nchunk = N // RC
        emax = (nchunk + NS - 1) // NS
        ed, wd = {}, {}

        e_r0s = [pl.multiple_of((sid + i * NS) * RC, 8) for i in range(emax)]
        col0 = pl.multiple_of(cid * CH, 128)

        def e_r0(i):
            return e_r0s[i]

        def e_pred(i):
            return sid + i * NS < nchunk

        def e_load(i):
            ed[i] = pltpu.async_copy(
                acc_sh.at[pl.ds(e_r0(i), RC)],
                rows_v.at[pl.ds((i % 2) * RC, RC)], sem_g)

        pl.when(e_pred(0))(lambda: e_load(0))
        for i in range(emax):
            @pl.when(e_pred(i))
            def _chunk(i=i):
                ed[i].wait()
                if i + 1 < emax:
                    @pl.when(e_pred(i + 1))
                    def _pre(i=i):
                        if i >= 1:
                            wd[i - 1].wait()
                        e_load(i + 1)
                pltpu.sync_copy(dis_hbm.at[pl.ds(e_r0(i), RC)], dis_c)

                def scale16(g, carry2):
                    dis16 = dis_c[pl.ds(g * 16, 16)]
                    for j in range(16):
                        row = (i % 2) * RC + g * 16 + j
                        s = dis16[j]
                        for k in range(CH // 16):
                            rows_v[row, pl.ds(k * 16, 16)] = (
                                rows_v[row, pl.ds(k * 16, 16)] * s)
                    return carry2
                lax.fori_loop(0, RC // 16, scale16, 0)
                wd[i] = pltpu.async_copy(
                    rows_v.at[pl.ds((i % 2) * RC, RC)],
                    out_hbm.at[pl.ds(e_r0(i), RC), pl.ds(col0, CH)],
                    sem_s)
        for i in range(emax):
            pred = e_pred(i) if i + 2 >= emax else (
                e_pred(i) & jnp.logical_not(e_pred(i + 2)))
            pl.when(pred)(lambda i=i: wd[i].wait())

    return sc_spmm


def kernel(x, edge_index_with_loops, deg_inv_sqrt, num_nodes, W):
    x = x.astype(jnp.float32)
    dis = deg_inv_sqrt.astype(jnp.float32)
    src = edge_index_with_loops[0].astype(jnp.int32)
    dst = edge_index_with_loops[1].astype(jnp.int32)
    e = src.shape[0]
    chunk = NS * EB
    e_pad = ((e + chunk - 1) // chunk) * chunk
    if e_pad != e:
        src = jnp.concatenate([src, jnp.zeros((e_pad - e,), jnp.int32)])
        dst = jnp.concatenate([dst, jnp.full((e_pad - e,), N, jnp.int32)])
    src2 = jnp.concatenate([src, src + N])
    y = _matmul_scaled(x, dis, W)
    return _make_sc_spmm(e_pad)(y, src2, dst, dis)


# speedup vs baseline: 1.0022x; 1.0022x over previous
import functools

import jax
import jax.numpy as jnp
from jax import lax
from jax.experimental import pallas as pl
from jax.experimental.pallas import tpu as pltpu
from jax.experimental.pallas import tpu_sc as plsc

N = 10000
C = 256
CH = 128
NC = 2
NS = 16
EB = 64
ACC = 10112
BM = 2000
NBUF = 5
IBUF = 7
GLA = 3
ILA = 5
RC = 80


def _mm_body(x_ref, d_ref, w_ref, o_ref):
    xs = x_ref[...] * d_ref[...]
    o_ref[...] = lax.dot_general(
        xs, w_ref[...], (((1,), (0,)), ((), ())),
        preferred_element_type=jnp.float32,
        precision=lax.Precision.DEFAULT)


def _matmul_scaled(x, dis, W):
    return pl.pallas_call(
        _mm_body,
        grid=(N // BM, NC),
        in_specs=[
            pl.BlockSpec((BM, C), lambda i, c: (i, 0)),
            pl.BlockSpec((BM, 1), lambda i, c: (i, 0)),
            pl.BlockSpec((C, CH), lambda i, c: (0, c)),
        ],
        out_specs=pl.BlockSpec((BM, CH), lambda i, c: (c * (N // BM) + i, 0)),
        out_shape=jax.ShapeDtypeStruct((NC * N, CH), jnp.float32),
    )(x, dis.reshape(N, 1), W)


def _make_sc_spmm(e_pad):
    per_tile = e_pad // NS
    nb = per_tile // EB
    mesh = plsc.VectorSubcoreMesh(core_axis_name="c", subcore_axis_name="s")

    @functools.partial(
        pl.kernel,
        out_type=jax.ShapeDtypeStruct((N, C), jnp.float32),
        mesh=mesh,
        scratch_types=[
            pltpu.VMEM((IBUF, EB), jnp.int32),
            pltpu.VMEM((IBUF, EB), jnp.int32),
            pltpu.VMEM((NBUF * EB, CH), jnp.float32),
            pltpu.VMEM((RC,), jnp.float32),
            pltpu.VMEM_SHARED((ACC, CH), jnp.float32),
            pltpu.SemaphoreType.DMA,
            pltpu.SemaphoreType.DMA,
            pltpu.SemaphoreType.DMA,
        ],
    )
    def sc_spmm(y_hbm, src_hbm, dst_hbm, dis_hbm, out_hbm,
                src_v, dst_v, rows_v, dis_c, acc_sh, sem_g, sem_s, sem_i):
        cid = lax.axis_index("c")
        sid = lax.axis_index("s")

        ebase = sid * per_tile
        soff = cid * e_pad
        idx, gd, sd = {}, {}, {}

        def start_idx(b):
            o = ebase + b * EB
            idx[b] = (
                pltpu.async_copy(src_hbm.at[pl.ds(soff + o, EB)],
                                 src_v.at[b % IBUF], sem_i),
                pltpu.async_copy(dst_hbm.at[pl.ds(o, EB)],
                                 dst_v.at[b % IBUF], sem_i))

        def start_gather(b):
            gd[b] = pltpu.async_copy(
                y_hbm.at[src_v.at[b % IBUF]],
                rows_v.at[pl.ds((b % NBUF) * EB, EB)], sem_g)

        def start_scatter(b):
            sd[b] = pltpu.async_copy(
                rows_v.at[pl.ds((b % NBUF) * EB, EB)],
                acc_sh.at[dst_v.at[b % IBUF]], sem_s, add=True)

        ZR = 128
        def zrow(r, carry):
            for k in range(CH // 16):
                rows_v[r, pl.ds(k * 16, 16)] = jnp.zeros((16,), jnp.float32)
            return carry
        lax.fori_loop(0, ZR, zrow, 0)
        arows = ACC // NS
        for j in range(arows // ZR):
            pltpu.sync_copy(rows_v.at[pl.ds(0, ZR)],
                            acc_sh.at[pl.ds(sid * arows + j * ZR, ZR)])
        rem = arows - (arows // ZR) * ZR
        pltpu.sync_copy(rows_v.at[pl.ds(0, rem)],
                        acc_sh.at[pl.ds(sid * arows + (arows // ZR) * ZR, rem)])

        for b in range(min(ILA, nb)):
            start_idx(b)
        for b in range(min(GLA, nb)):
            idx[b][0].wait()
            idx[b][1].wait()
            start_gather(b)
        plsc.subcore_barrier()

        for b in range(nb):
            if b >= 2:
                sd[b - 2].wait()
            if b + GLA < nb:
                idx[b + GLA][0].wait()
                idx[b + GLA][1].wait()
                start_gather(b + GLA)
            if b + ILA < nb:
                start_idx(b + ILA)
            gd[b].wait()
            start_scatter(b)
        for b in range(max(0, nb - 2), nb):
            sd[b].wait()

        plsc.subcore_barrier()

        nchunk = N // RC
        emax = (nchunk + NS - 1) // NS
        ed, wd = {}, {}

        e_r0s = [pl.multiple_of((sid + i * NS) * RC, 8) for i in range(emax)]
        col0 = pl.multiple_of(cid * CH, 128)

        def e_r0(i):
            return e_r0s[i]

        def e_pred(i):
            return sid + i * NS < nchunk

        def e_load(i):
            ed[i] = pltpu.async_copy(
                acc_sh.at[pl.ds(e_r0(i), RC)],
                rows_v.at[pl.ds((i % 2) * RC, RC)], sem_g)

        pl.when(e_pred(0))(lambda: e_load(0))
        for i in range(emax):
            @pl.when(e_pred(i))
            def _chunk(i=i):
                ed[i].wait()
                if i + 1 < emax:
                    @pl.when(e_pred(i + 1))
                    def _pre(i=i):
                        if i >= 1:
                            wd[i - 1].wait()
                        e_load(i + 1)
                pltpu.sync_copy(dis_hbm.at[pl.ds(e_r0(i), RC)], dis_c)

                def scale16(g, carry2):
                    dis16 = dis_c[pl.ds(g * 16, 16)]
                    for j in range(16):
                        row = (i % 2) * RC + g * 16 + j
                        s = dis16[j]
                        for k in range(CH // 16):
                            rows_v[row, pl.ds(k * 16, 16)] = (
                                rows_v[row, pl.ds(k * 16, 16)] * s)
                    return carry2
                lax.fori_loop(0, RC // 16, scale16, 0)
                wd[i] = pltpu.async_copy(
                    rows_v.at[pl.ds((i % 2) * RC, RC)],
                    out_hbm.at[pl.ds(e_r0(i), RC), pl.ds(col0, CH)],
                    sem_s)
        for i in range(emax):
            pred = e_pred(i) if i + 2 >= emax else (
                e_pred(i) & jnp.logical_not(e_pred(i + 2)))
            pl.when(pred)(lambda i=i: wd[i].wait())

    return sc_spmm


def kernel(x, edge_index_with_loops, deg_inv_sqrt, num_nodes, W):
    x = x.astype(jnp.float32)
    dis = deg_inv_sqrt.astype(jnp.float32)
    src = edge_index_with_loops[0].astype(jnp.int32)
    dst = edge_index_with_loops[1].astype(jnp.int32)
    e = src.shape[0]
    chunk = NS * EB
    e_pad = ((e + chunk - 1) // chunk) * chunk
    if e_pad != e:
        src = jnp.concatenate([src, jnp.zeros((e_pad - e,), jnp.int32)])
        dst = jnp.concatenate([dst, jnp.full((e_pad - e,), N, jnp.int32)])
    src2 = jnp.concatenate([src, src + N])
    y = _matmul_scaled(x, dis, W)
    return _make_sc_spmm(e_pad)(y, src2, dst, dis)
